# baseline (speedup 1.0000x reference)
import jax
import jax.numpy as jnp
from jax.experimental import pallas as pl
from jax.experimental.pallas import tpu as pltpu

_C = 80
_HW = 25600
_NCHUNK = 200
_TOPK = 1000
_POSTK = 100
_NEG = -3.4e38
_BIGI = 2**30


def _tree(x, op):
    for s in (1, 2, 4, 8, 16, 32, 64):
        x = op(x, pltpu.roll(x, s, 1))
    r = 1
    while r < x.shape[0]:
        x = op(x, pltpu.roll(x, r, 0))
        r *= 2
    return x


def _smax(x):
    return _tree(x, jnp.maximum)


def _smin(x):
    return _tree(x, jnp.minimum)


def _ssum(x):
    return _tree(x, jnp.add)


def _bc16(x):
    return jnp.broadcast_to(x[0:1, :], (16, 128))


def _fcos_kernel(cls_ref, ctr_ref, reg_ref, loc_ref, bT_ref, sc_ref, cl_ref,
                 scores_ref):
    i32 = jnp.int32
    r8 = jax.lax.broadcasted_iota(i32, (8, 128), 0)
    c8 = jax.lax.broadcasted_iota(i32, (8, 128), 1)
    flat = r8 * 128 + c8
    r16 = jax.lax.broadcasted_iota(i32, (16, 128), 0)
    c16 = jax.lax.broadcasted_iota(i32, (16, 128), 1)
    flat16 = r16 * 128 + c16
    col4 = jax.lax.broadcasted_iota(i32, (4, 128), 1)
    col2 = jax.lax.broadcasted_iota(i32, (2, 128), 1)

    def p1_body(c, carry):
        cmax, carg = carry
        base = pl.multiple_of(c * 128, 128)
        sg = jax.nn.sigmoid(cls_ref[:, pl.ds(base, 128)])
        ct = jax.nn.sigmoid(ctr_ref[:, pl.ds(base, 128)])
        s = jnp.where(sg > 0.05, sg * ct, -1.0)
        scores_ref[:, pl.ds(base, 128)] = s
        for rb in range(10):
            sub = s[rb * 8:(rb + 1) * 8, :]
            msub = _smax(sub)
            ridx = (base + c8) * _C + (rb * 8) + r8
            asub = _smin(jnp.where(sub == msub, ridx, _BIGI))
            hit = flat16 == (rb * _NCHUNK + c)
            cmax = jnp.where(hit, _bc16(msub), cmax)
            carg = jnp.where(hit, _bc16(asub), carg)
        return cmax, carg

    cmax0 = jnp.full((16, 128), -3.0, jnp.float32)
    carg0 = jnp.full((16, 128), _BIGI, i32)
    cmax, carg = jax.lax.fori_loop(0, _NCHUNK, p1_body, (cmax0, carg0))

    def p2_body(t, carry):
        cmax, carg, ax1, ay1, ax2, ay2, asc, acl = carry
        m16 = _smax(cmax)
        r = jnp.min(jnp.where(cmax == m16, carg, _BIGI))
        loc = r // _C
        klass = r - loc * _C
        rb = klass // 8
        rowb = pl.multiple_of(rb * 8, 8)
        cc = loc // 128
        base = pl.multiple_of(cc * 128, 128)
        colin = loc - base
        blk = scores_ref[pl.ds(rowb, 8), pl.ds(base, 128)]
        clr = jnp.where((r8 == klass - rowb) & (c8 == colin), -2.0, blk)
        scores_ref[pl.ds(rowb, 8), pl.ds(base, 128)] = clr
        nm = _smax(clr)
        ridx = (base + c8) * _C + rowb + r8
        nr = _smin(jnp.where(clr == nm, ridx, _BIGI))
        hit = flat16 == (rb * _NCHUNK + cc)
        cmax = jnp.where(hit, _bc16(nm), cmax)
        carg = jnp.where(hit, _bc16(nr), carg)
        rsel = jnp.where(col4 == colin, reg_ref[:, pl.ds(base, 128)], 0.0)
        lsel = jnp.where(col2 == colin, loc_ref[:, pl.ds(base, 128)], 0.0)
        g0 = _ssum(rsel[0:1, :])
        g1 = _ssum(rsel[1:2, :])
        g2 = _ssum(rsel[2:3, :])
        g3 = _ssum(rsel[3:4, :])
        lx = _ssum(lsel[0:1, :])
        ly = _ssum(lsel[1:2, :])
        pm = flat == t
        m8 = m16[0:8, :]
        ax1 = jnp.where(pm, lx - g0, ax1)
        ay1 = jnp.where(pm, ly - g1, ay1)
        ax2 = jnp.where(pm, lx + g2, ax2)
        ay2 = jnp.where(pm, ly + g3, ay2)
        asc = jnp.where(pm, jnp.sqrt(m8), asc)
        acl = jnp.where(pm, klass, acl)
        return cmax, carg, ax1, ay1, ax2, ay2, asc, acl

    zf = jnp.zeros((8, 128), jnp.float32)
    init = (cmax, carg, zf, zf, zf, zf, jnp.full((8, 128), -1.0, jnp.float32),
            jnp.zeros((8, 128), i32))
    (_, _, ax1, ay1, ax2, ay2, asc, acl) = jax.lax.fori_loop(
        0, _TOPK, p2_body, init)

    valid = flat < _TOPK
    mc = jnp.max(jnp.where(
        valid,
        jnp.maximum(jnp.maximum(ax1, ax2), jnp.maximum(ay1, ay2)), _NEG))
    off = acl.astype(jnp.float32) * (mc + 1.0)
    bx1 = ax1 + off
    by1 = ay1 + off
    bx2 = ax2 + off
    by2 = ay2 + off
    area = jnp.maximum(bx2 - bx1, 0.0) * jnp.maximum(by2 - by1, 0.0)

    def nms_body(pos, supp):
        pm = flat == pos
        px1 = _smax(jnp.where(pm, bx1, _NEG))
        py1 = _smax(jnp.where(pm, by1, _NEG))
        px2 = _smax(jnp.where(pm, bx2, _NEG))
        py2 = _smax(jnp.where(pm, by2, _NEG))
        pa = _smax(jnp.where(pm, area, _NEG))
        alive = _smax(jnp.where(pm, supp, 0)) == 0
        ix1 = jnp.maximum(px1, bx1)
        iy1 = jnp.maximum(py1, by1)
        ix2 = jnp.minimum(px2, bx2)
        iy2 = jnp.minimum(py2, by2)
        inter = jnp.maximum(ix2 - ix1, 0.0) * jnp.maximum(iy2 - iy1, 0.0)
        iou = inter / (pa + area - inter + 1e-9)
        supnew = alive & (flat > pos) & (iou > 0.6)
        return jnp.where(supnew, 1, supp)

    supp = jax.lax.fori_loop(0, _TOPK, nms_body,
                             jnp.where(valid, 0, 1).astype(i32))
    kept = jnp.where((supp == 0) & valid, 1, 0).astype(i32)

    lane128 = jax.lax.broadcasted_iota(i32, (1, 128), 1)

    def sel_body(t, carry):
        keptv, ox1, oy1, ox2, oy2, osc, ocl = carry
        p = _smin(jnp.where(keptv == 1, flat, 1024))
        p = jnp.where(p == 1024, _TOPK - 1, p)
        pm = flat == p
        gx1 = _smax(jnp.where(pm, ax1, _NEG))
        gy1 = _smax(jnp.where(pm, ay1, _NEG))
        gx2 = _smax(jnp.where(pm, ax2, _NEG))
        gy2 = _smax(jnp.where(pm, ay2, _NEG))
        gsc = _smax(jnp.where(pm, asc, _NEG))
        gcl = _smax(jnp.where(pm, acl, 0))
        tm = lane128 == t
        ox1 = jnp.where(tm, gx1[0:1, :], ox1)
        oy1 = jnp.where(tm, gy1[0:1, :], oy1)
        ox2 = jnp.where(tm, gx2[0:1, :], ox2)
        oy2 = jnp.where(tm, gy2[0:1, :], oy2)
        osc = jnp.where(tm, gsc[0:1, :], osc)
        ocl = jnp.where(tm, gcl[0:1, :], ocl)
        keptv = jnp.where(pm, 0, keptv)
        return keptv, ox1, oy1, ox2, oy2, osc, ocl

    zl = jnp.zeros((1, 128), jnp.float32)
    (_, ox1, oy1, ox2, oy2, osc, ocl) = jax.lax.fori_loop(
        0, _POSTK, sel_body,
        (kept, zl, zl, zl, zl, zl, jnp.zeros((1, 128), i32)))

    bT_ref[0:1, :] = ox1
    bT_ref[1:2, :] = oy1
    bT_ref[2:3, :] = ox2
    bT_ref[3:4, :] = oy2
    sc_ref[:, :] = osc
    cl_ref[:, :] = ocl


def _call(cls2d, ctr2d, reg2d, locT, interpret=False):
    return pl.pallas_call(
        _fcos_kernel,
        out_shape=[
            jax.ShapeDtypeStruct((4, 128), jnp.float32),
            jax.ShapeDtypeStruct((1, 128), jnp.float32),
            jax.ShapeDtypeStruct((1, 128), jnp.int32),
        ],
        scratch_shapes=[pltpu.VMEM((_C, _HW), jnp.float32)],
        interpret=interpret,
    )(cls2d, ctr2d, reg2d, locT)


@jax.jit
def kernel(locations, box_cls, box_regression, centerness):
    cls2d = box_cls.reshape(_C, _HW)
    reg2d = box_regression.reshape(4, _HW)
    ctr2d = centerness.reshape(1, _HW)
    locT = locations.T
    bT, sc, cl = _call(cls2d, ctr2d, reg2d, locT)
    boxes = bT[:, :_POSTK].T
    return boxes, sc[0, :_POSTK], cl[0, :_POSTK]


# speedup vs baseline: 1.8762x; 1.8762x over previous
import jax
import jax.numpy as jnp
from jax.experimental import pallas as pl
from jax.experimental.pallas import tpu as pltpu

_C = 80
_HW = 25600
_NCHUNK = 200
_TOPK = 1000
_POSTK = 100
_NEG = -3.4e38
_BIGI = 2**30


def _fcos_kernel(cls_ref, ctr_ref, reg_ref, loc_ref, bT_ref, sc_ref, cl_ref,
                 scores_ref):
    i32 = jnp.int32
    col4 = jax.lax.broadcasted_iota(i32, (4, 128), 1)
    col2 = jax.lax.broadcasted_iota(i32, (2, 128), 1)
    r8 = jax.lax.broadcasted_iota(i32, (8, 128), 0)
    c8 = jax.lax.broadcasted_iota(i32, (8, 128), 1)
    flat = r8 * 128 + c8
    r16 = jax.lax.broadcasted_iota(i32, (16, 128), 0)
    c16 = jax.lax.broadcasted_iota(i32, (16, 128), 1)
    flat16 = r16 * 128 + c16

    def p1_body(c, carry):
        cmax, carg = carry
        base = pl.multiple_of(c * 128, 128)
        sg = jax.nn.sigmoid(cls_ref[:, pl.ds(base, 128)])
        ct = jax.nn.sigmoid(ctr_ref[:, pl.ds(base, 128)])
        s = jnp.where(sg > 0.05, sg * ct, -1.0)
        scores_ref[:, pl.ds(base, 128)] = s
        for rb in range(10):
            sub = s[rb * 8:(rb + 1) * 8, :]
            m = jnp.max(sub)
            ridx = (base + c8) * _C + (rb * 8) + r8
            rmin = jnp.min(jnp.where(sub == m, ridx, _BIGI))
            hit = flat16 == (rb * _NCHUNK + c)
            cmax = jnp.where(hit, m, cmax)
            carg = jnp.where(hit, rmin, carg)
        return cmax, carg

    cmax0 = jnp.full((16, 128), -3.0, jnp.float32)
    carg0 = jnp.full((16, 128), _BIGI, i32)
    cmax, carg = jax.lax.fori_loop(0, _NCHUNK, p1_body, (cmax0, carg0))

    def p2_body(t, carry):
        cmax, carg, ax1, ay1, ax2, ay2, asc, acl = carry
        m = jnp.max(cmax)
        r = jnp.min(jnp.where(cmax == m, carg, _BIGI))
        loc = r // _C
        klass = r - loc * _C
        rb = klass // 8
        rowb = pl.multiple_of(rb * 8, 8)
        cc = loc // 128
        base = pl.multiple_of(cc * 128, 128)
        colin = loc - base
        blk = scores_ref[pl.ds(rowb, 8), pl.ds(base, 128)]
        clr = jnp.where((r8 == klass - rowb) & (c8 == colin), -2.0, blk)
        scores_ref[pl.ds(rowb, 8), pl.ds(base, 128)] = clr
        nm = jnp.max(clr)
        ridx = (base + c8) * _C + rowb + r8
        nr = jnp.min(jnp.where(clr == nm, ridx, _BIGI))
        hit = flat16 == (rb * _NCHUNK + cc)
        cmax = jnp.where(hit, nm, cmax)
        carg = jnp.where(hit, nr, carg)
        rsel = jnp.where(col4 == colin, reg_ref[:, pl.ds(base, 128)], 0.0)
        lsel = jnp.where(col2 == colin, loc_ref[:, pl.ds(base, 128)], 0.0)
        r0 = jnp.sum(rsel[0:1, :])
        r1 = jnp.sum(rsel[1:2, :])
        r2 = jnp.sum(rsel[2:3, :])
        r3 = jnp.sum(rsel[3:4, :])
        lx = jnp.sum(lsel[0:1, :])
        ly = jnp.sum(lsel[1:2, :])
        pm = flat == t
        ax1 = jnp.where(pm, lx - r0, ax1)
        ay1 = jnp.where(pm, ly - r1, ay1)
        ax2 = jnp.where(pm, lx + r2, ax2)
        ay2 = jnp.where(pm, ly + r3, ay2)
        asc = jnp.where(pm, jnp.sqrt(m), asc)
        acl = jnp.where(pm, klass, acl)
        return cmax, carg, ax1, ay1, ax2, ay2, asc, acl

    zf = jnp.zeros((8, 128), jnp.float32)
    init = (cmax, carg, zf, zf, zf, zf, jnp.full((8, 128), -1.0, jnp.float32),
            jnp.zeros((8, 128), i32))
    (_, _, ax1, ay1, ax2, ay2, asc, acl) = jax.lax.fori_loop(
        0, _TOPK, p2_body, init)

    valid = flat < _TOPK
    mc = jnp.max(jnp.where(
        valid,
        jnp.maximum(jnp.maximum(ax1, ax2), jnp.maximum(ay1, ay2)), _NEG))
    off = acl.astype(jnp.float32) * (mc + 1.0)
    bx1 = ax1 + off
    by1 = ay1 + off
    bx2 = ax2 + off
    by2 = ay2 + off
    area = jnp.maximum(bx2 - bx1, 0.0) * jnp.maximum(by2 - by1, 0.0)

    def nms_body(pos, supp):
        pm = flat == pos
        px1 = jnp.max(jnp.where(pm, bx1, _NEG))
        py1 = jnp.max(jnp.where(pm, by1, _NEG))
        px2 = jnp.max(jnp.where(pm, bx2, _NEG))
        py2 = jnp.max(jnp.where(pm, by2, _NEG))
        pa = jnp.max(jnp.where(pm, area, _NEG))
        alive = jnp.max(jnp.where(pm, supp, 0)) == 0
        ix1 = jnp.maximum(px1, bx1)
        iy1 = jnp.maximum(py1, by1)
        ix2 = jnp.minimum(px2, bx2)
        iy2 = jnp.minimum(py2, by2)
        inter = jnp.maximum(ix2 - ix1, 0.0) * jnp.maximum(iy2 - iy1, 0.0)
        iou = inter / (pa + area - inter + 1e-9)
        supnew = alive & (flat > pos) & (iou > 0.6)
        return jnp.where(supnew, 1, supp)

    supp = jax.lax.fori_loop(0, _TOPK, nms_body,
                             jnp.where(valid, 0, 1).astype(i32))
    kept = jnp.where((supp == 0) & valid, 1, 0).astype(i32)

    lane128 = jax.lax.broadcasted_iota(i32, (1, 128), 1)

    def sel_body(t, carry):
        keptv, ox1, oy1, ox2, oy2, osc, ocl = carry
        p = jnp.min(jnp.where(keptv == 1, flat, 1024))
        p = jnp.where(p == 1024, _TOPK - 1, p)
        pm = flat == p
        gx1 = jnp.max(jnp.where(pm, ax1, _NEG))
        gy1 = jnp.max(jnp.where(pm, ay1, _NEG))
        gx2 = jnp.max(jnp.where(pm, ax2, _NEG))
        gy2 = jnp.max(jnp.where(pm, ay2, _NEG))
        gsc = jnp.max(jnp.where(pm, asc, _NEG))
        gcl = jnp.max(jnp.where(pm, acl, 0))
        tm = lane128 == t
        ox1 = jnp.where(tm, gx1, ox1)
        oy1 = jnp.where(tm, gy1, oy1)
        ox2 = jnp.where(tm, gx2, ox2)
        oy2 = jnp.where(tm, gy2, oy2)
        osc = jnp.where(tm, gsc, osc)
        ocl = jnp.where(tm, gcl, ocl)
        keptv = jnp.where(pm, 0, keptv)
        return keptv, ox1, oy1, ox2, oy2, osc, ocl

    zl = jnp.zeros((1, 128), jnp.float32)
    (_, ox1, oy1, ox2, oy2, osc, ocl) = jax.lax.fori_loop(
        0, _POSTK, sel_body,
        (kept, zl, zl, zl, zl, zl, jnp.zeros((1, 128), i32)))

    bT_ref[0:1, :] = ox1
    bT_ref[1:2, :] = oy1
    bT_ref[2:3, :] = ox2
    bT_ref[3:4, :] = oy2
    sc_ref[:, :] = osc
    cl_ref[:, :] = ocl


def _call(cls2d, ctr2d, reg2d, locT, interpret=False):
    return pl.pallas_call(
        _fcos_kernel,
        out_shape=[
            jax.ShapeDtypeStruct((4, 128), jnp.float32),
            jax.ShapeDtypeStruct((1, 128), jnp.float32),
            jax.ShapeDtypeStruct((1, 128), jnp.int32),
        ],
        scratch_shapes=[pltpu.VMEM((_C, _HW), jnp.float32)],
        interpret=interpret,
    )(cls2d, ctr2d, reg2d, locT)


@jax.jit
def kernel(locations, box_cls, box_regression, centerness):
    cls2d = box_cls.reshape(_C, _HW)
    reg2d = box_regression.reshape(4, _HW)
    ctr2d = centerness.reshape(1, _HW)
    locT = locations.T
    bT, sc, cl = _call(cls2d, ctr2d, reg2d, locT)
    boxes = bT[:, :_POSTK].T
    return boxes, sc[0, :_POSTK], cl[0, :_POSTK]


# speedup vs baseline: 2.7601x; 1.4711x over previous
import jax
import jax.numpy as jnp
from jax.experimental import pallas as pl
from jax.experimental.pallas import tpu as pltpu

_C = 80
_HW = 25600
_NCHUNK = 200
_TOPK = 1000
_POSTK = 100
_NEG = -3.4e38
_BIGI = 2**30


def _fcos_kernel(cls_ref, ctr_ref, reg_ref, loc_ref, bT_ref, sc_ref, cl_ref,
                 scores_ref):
    i32 = jnp.int32
    row80 = jax.lax.broadcasted_iota(i32, (_C, 128), 0)
    col80 = jax.lax.broadcasted_iota(i32, (_C, 128), 1)
    lane256 = jax.lax.broadcasted_iota(i32, (1, 256), 1)
    col4 = jax.lax.broadcasted_iota(i32, (4, 128), 1)
    col2 = jax.lax.broadcasted_iota(i32, (2, 128), 1)
    r8 = jax.lax.broadcasted_iota(i32, (8, 128), 0)
    c8 = jax.lax.broadcasted_iota(i32, (8, 128), 1)
    flat = r8 * 128 + c8

    def p1_body(c, carry):
        cmax, carg = carry
        base = pl.multiple_of(c * 128, 128)
        sg = jax.nn.sigmoid(cls_ref[:, pl.ds(base, 128)])
        ct = jax.nn.sigmoid(ctr_ref[:, pl.ds(base, 128)])
        s = jnp.where(sg > 0.05, sg * ct, -1.0)
        scores_ref[:, pl.ds(base, 128)] = s
        m = jnp.max(s)
        ridx = (base + col80) * _C + row80
        rmin = jnp.min(jnp.where(s == m, ridx, _BIGI))
        hit = lane256 == c
        return jnp.where(hit, m, cmax), jnp.where(hit, rmin, carg)

    cmax0 = jnp.full((1, 256), -3.0, jnp.float32)
    carg0 = jnp.full((1, 256), _BIGI, i32)
    cmax, carg = jax.lax.fori_loop(0, _NCHUNK, p1_body, (cmax0, carg0))

    def p2_body(t, carry):
        cmax, carg, ax1, ay1, ax2, ay2, asc, acl = carry
        m = jnp.max(cmax)
        r = jnp.min(jnp.where(cmax == m, carg, _BIGI))
        loc = r // _C
        klass = r - loc * _C
        cc = loc // 128
        base = pl.multiple_of(cc * 128, 128)
        colin = loc - base
        blk = scores_ref[:, pl.ds(base, 128)]
        clr = jnp.where((row80 == klass) & (col80 == colin), -2.0, blk)
        scores_ref[:, pl.ds(base, 128)] = clr
        nm = jnp.max(clr)
        ridx = (base + col80) * _C + row80
        nr = jnp.min(jnp.where(clr == nm, ridx, _BIGI))
        hit = lane256 == cc
        cmax = jnp.where(hit, nm, cmax)
        carg = jnp.where(hit, nr, carg)
        rsel = jnp.where(col4 == colin, reg_ref[:, pl.ds(base, 128)], 0.0)
        lsel = jnp.where(col2 == colin, loc_ref[:, pl.ds(base, 128)], 0.0)
        r0 = jnp.sum(rsel[0:1, :])
        r1 = jnp.sum(rsel[1:2, :])
        r2 = jnp.sum(rsel[2:3, :])
        r3 = jnp.sum(rsel[3:4, :])
        lx = jnp.sum(lsel[0:1, :])
        ly = jnp.sum(lsel[1:2, :])
        pm = flat == t
        ax1 = jnp.where(pm, lx - r0, ax1)
        ay1 = jnp.where(pm, ly - r1, ay1)
        ax2 = jnp.where(pm, lx + r2, ax2)
        ay2 = jnp.where(pm, ly + r3, ay2)
        asc = jnp.where(pm, jnp.sqrt(m), asc)
        acl = jnp.where(pm, klass, acl)
        return cmax, carg, ax1, ay1, ax2, ay2, asc, acl

    def p2_body4(i, carry):
        carry = p2_body(4 * i, carry)
        carry = p2_body(4 * i + 1, carry)
        carry = p2_body(4 * i + 2, carry)
        carry = p2_body(4 * i + 3, carry)
        return carry

    zf = jnp.zeros((8, 128), jnp.float32)
    init = (cmax, carg, zf, zf, zf, zf, jnp.full((8, 128), -1.0, jnp.float32),
            jnp.zeros((8, 128), i32))
    (_, _, ax1, ay1, ax2, ay2, asc, acl) = jax.lax.fori_loop(
        0, _TOPK // 4, p2_body4, init)

    valid = flat < _TOPK
    mc = jnp.max(jnp.where(
        valid,
        jnp.maximum(jnp.maximum(ax1, ax2), jnp.maximum(ay1, ay2)), _NEG))
    off = acl.astype(jnp.float32) * (mc + 1.0)
    bx1 = ax1 + off
    by1 = ay1 + off
    bx2 = ax2 + off
    by2 = ay2 + off
    area = jnp.maximum(bx2 - bx1, 0.0) * jnp.maximum(by2 - by1, 0.0)

    def nms_body(pos, supp):
        pm = flat == pos
        px1 = jnp.max(jnp.where(pm, bx1, _NEG))
        py1 = jnp.max(jnp.where(pm, by1, _NEG))
        px2 = jnp.max(jnp.where(pm, bx2, _NEG))
        py2 = jnp.max(jnp.where(pm, by2, _NEG))
        pa = jnp.max(jnp.where(pm, area, _NEG))
        alive = jnp.max(jnp.where(pm, supp, 0)) == 0
        ix1 = jnp.maximum(px1, bx1)
        iy1 = jnp.maximum(py1, by1)
        ix2 = jnp.minimum(px2, bx2)
        iy2 = jnp.minimum(py2, by2)
        inter = jnp.maximum(ix2 - ix1, 0.0) * jnp.maximum(iy2 - iy1, 0.0)
        iou = inter / (pa + area - inter + 1e-9)
        supnew = alive & (flat > pos) & (iou > 0.6)
        return jnp.where(supnew, 1, supp)

    supp = jax.lax.fori_loop(0, _TOPK, nms_body,
                             jnp.where(valid, 0, 1).astype(i32))
    kept = jnp.where((supp == 0) & valid, 1, 0).astype(i32)

    lane128 = jax.lax.broadcasted_iota(i32, (1, 128), 1)

    def sel_body(t, carry):
        keptv, ox1, oy1, ox2, oy2, osc, ocl = carry
        p = jnp.min(jnp.where(keptv == 1, flat, 1024))
        p = jnp.where(p == 1024, _TOPK - 1, p)
        pm = flat == p
        gx1 = jnp.max(jnp.where(pm, ax1, _NEG))
        gy1 = jnp.max(jnp.where(pm, ay1, _NEG))
        gx2 = jnp.max(jnp.where(pm, ax2, _NEG))
        gy2 = jnp.max(jnp.where(pm, ay2, _NEG))
        gsc = jnp.max(jnp.where(pm, asc, _NEG))
        gcl = jnp.max(jnp.where(pm, acl, 0))
        tm = lane128 == t
        ox1 = jnp.where(tm, gx1, ox1)
        oy1 = jnp.where(tm, gy1, oy1)
        ox2 = jnp.where(tm, gx2, ox2)
        oy2 = jnp.where(tm, gy2, oy2)
        osc = jnp.where(tm, gsc, osc)
        ocl = jnp.where(tm, gcl, ocl)
        keptv = jnp.where(pm, 0, keptv)
        return keptv, ox1, oy1, ox2, oy2, osc, ocl

    zl = jnp.zeros((1, 128), jnp.float32)
    (_, ox1, oy1, ox2, oy2, osc, ocl) = jax.lax.fori_loop(
        0, _POSTK, sel_body,
        (kept, zl, zl, zl, zl, zl, jnp.zeros((1, 128), i32)))

    bT_ref[0:1, :] = ox1
    bT_ref[1:2, :] = oy1
    bT_ref[2:3, :] = ox2
    bT_ref[3:4, :] = oy2
    sc_ref[:, :] = osc
    cl_ref[:, :] = ocl


def _call(cls2d, ctr2d, reg2d, locT, interpret=False):
    return pl.pallas_call(
        _fcos_kernel,
        out_shape=[
            jax.ShapeDtypeStruct((4, 128), jnp.float32),
            jax.ShapeDtypeStruct((1, 128), jnp.float32),
            jax.ShapeDtypeStruct((1, 128), jnp.int32),
        ],
        scratch_shapes=[pltpu.VMEM((_C, _HW), jnp.float32)],
        interpret=interpret,
    )(cls2d, ctr2d, reg2d, locT)


@jax.jit
def kernel(locations, box_cls, box_regression, centerness):
    cls2d = box_cls.reshape(_C, _HW)
    reg2d = box_regression.reshape(4, _HW)
    ctr2d = centerness.reshape(1, _HW)
    locT = locations.T
    bT, sc, cl = _call(cls2d, ctr2d, reg2d, locT)
    boxes = bT[:, :_POSTK].T
    return boxes, sc[0, :_POSTK], cl[0, :_POSTK]


# speedup vs baseline: 2.8167x; 1.0205x over previous
import jax
import jax.numpy as jnp
from jax.experimental import pallas as pl
from jax.experimental.pallas import tpu as pltpu

_C = 80
_HW = 25600
_NCHUNK = 200
_TOPK = 1000
_POSTK = 100
_NEG = -3.4e38
_BIGI = 2**30


def _fcos_kernel(cls_ref, ctr_ref, reg_ref, loc_ref, bT_ref, sc_ref, cl_ref,
                 scores_ref):
    i32 = jnp.int32
    row80 = jax.lax.broadcasted_iota(i32, (_C, 128), 0)
    col80 = jax.lax.broadcasted_iota(i32, (_C, 128), 1)
    lane256 = jax.lax.broadcasted_iota(i32, (1, 256), 1)
    col4 = jax.lax.broadcasted_iota(i32, (4, 128), 1)
    col2 = jax.lax.broadcasted_iota(i32, (2, 128), 1)
    r8 = jax.lax.broadcasted_iota(i32, (8, 128), 0)
    c8 = jax.lax.broadcasted_iota(i32, (8, 128), 1)
    flat = r8 * 128 + c8

    def p1_body(c, carry):
        cmax, carg = carry
        base = pl.multiple_of(c * 128, 128)
        sg = jax.nn.sigmoid(cls_ref[:, pl.ds(base, 128)])
        ct = jax.nn.sigmoid(ctr_ref[:, pl.ds(base, 128)])
        s = jnp.where(sg > 0.05, sg * ct, -1.0)
        scores_ref[:, pl.ds(base, 128)] = s
        m = jnp.max(s)
        ridx = (base + col80) * _C + row80
        rmin = jnp.min(jnp.where(s == m, ridx, _BIGI))
        hit = lane256 == c
        return jnp.where(hit, m, cmax), jnp.where(hit, rmin, carg)

    cmax0 = jnp.full((1, 256), -3.0, jnp.float32)
    carg0 = jnp.full((1, 256), _BIGI, i32)
    cmax, carg = jax.lax.fori_loop(0, _NCHUNK, p1_body, (cmax0, carg0))

    def p2_body(t, carry):
        cmax, carg, ax1, ay1, ax2, ay2, asc, acl = carry
        m = jnp.max(cmax)
        r = jnp.min(jnp.where(cmax == m, carg, _BIGI))
        loc = r // _C
        klass = r - loc * _C
        cc = loc // 128
        base = pl.multiple_of(cc * 128, 128)
        colin = loc - base
        blk = scores_ref[:, pl.ds(base, 128)]
        clr = jnp.where((row80 == klass) & (col80 == colin), -2.0, blk)
        scores_ref[:, pl.ds(base, 128)] = clr
        nm = jnp.max(clr)
        ridx = (base + col80) * _C + row80
        nr = jnp.min(jnp.where(clr == nm, ridx, _BIGI))
        hit = lane256 == cc
        cmax = jnp.where(hit, nm, cmax)
        carg = jnp.where(hit, nr, carg)
        rsel = jnp.where(col4 == colin, reg_ref[:, pl.ds(base, 128)], 0.0)
        lsel = jnp.where(col2 == colin, loc_ref[:, pl.ds(base, 128)], 0.0)
        r0 = jnp.sum(rsel[0:1, :])
        r1 = jnp.sum(rsel[1:2, :])
        r2 = jnp.sum(rsel[2:3, :])
        r3 = jnp.sum(rsel[3:4, :])
        lx = jnp.sum(lsel[0:1, :])
        ly = jnp.sum(lsel[1:2, :])
        pm = flat == t
        ax1 = jnp.where(pm, lx - r0, ax1)
        ay1 = jnp.where(pm, ly - r1, ay1)
        ax2 = jnp.where(pm, lx + r2, ax2)
        ay2 = jnp.where(pm, ly + r3, ay2)
        asc = jnp.where(pm, jnp.sqrt(m), asc)
        acl = jnp.where(pm, klass, acl)
        return cmax, carg, ax1, ay1, ax2, ay2, asc, acl

    def p2_body4(i, carry):
        carry = p2_body(4 * i, carry)
        carry = p2_body(4 * i + 1, carry)
        carry = p2_body(4 * i + 2, carry)
        carry = p2_body(4 * i + 3, carry)
        return carry

    zf = jnp.zeros((8, 128), jnp.float32)
    init = (cmax, carg, zf, zf, zf, zf, jnp.full((8, 128), -1.0, jnp.float32),
            jnp.zeros((8, 128), i32))
    (_, _, ax1, ay1, ax2, ay2, asc, acl) = jax.lax.fori_loop(
        0, _TOPK // 4, p2_body4, init)

    valid = flat < _TOPK
    mc = jnp.max(jnp.where(
        valid,
        jnp.maximum(jnp.maximum(ax1, ax2), jnp.maximum(ay1, ay2)), _NEG))
    off = acl.astype(jnp.float32) * (mc + 1.0)
    bx1 = ax1 + off
    by1 = ay1 + off
    bx2 = ax2 + off
    by2 = ay2 + off
    area = jnp.maximum(bx2 - bx1, 0.0) * jnp.maximum(by2 - by1, 0.0)

    def nms_body(pos, supp):
        pm = flat == pos
        px1 = jnp.max(jnp.where(pm, bx1, _NEG))
        py1 = jnp.max(jnp.where(pm, by1, _NEG))
        px2 = jnp.max(jnp.where(pm, bx2, _NEG))
        py2 = jnp.max(jnp.where(pm, by2, _NEG))
        pa = jnp.max(jnp.where(pm, area, _NEG))
        alive = jnp.max(jnp.where(pm, supp, 0)) == 0
        ix1 = jnp.maximum(px1, bx1)
        iy1 = jnp.maximum(py1, by1)
        ix2 = jnp.minimum(px2, bx2)
        iy2 = jnp.minimum(py2, by2)
        inter = jnp.maximum(ix2 - ix1, 0.0) * jnp.maximum(iy2 - iy1, 0.0)
        iou = inter / (pa + area - inter + 1e-9)
        supnew = alive & (flat > pos) & (iou > 0.6)
        return jnp.where(supnew, 1, supp)

    def nms_body4(i, supp):
        supp = nms_body(4 * i, supp)
        supp = nms_body(4 * i + 1, supp)
        supp = nms_body(4 * i + 2, supp)
        supp = nms_body(4 * i + 3, supp)
        return supp

    supp = jax.lax.fori_loop(0, _TOPK // 4, nms_body4,
                             jnp.where(valid, 0, 1).astype(i32))
    kept = jnp.where((supp == 0) & valid, 1, 0).astype(i32)

    lane128 = jax.lax.broadcasted_iota(i32, (1, 128), 1)

    def sel_body(t, carry):
        keptv, ox1, oy1, ox2, oy2, osc, ocl = carry
        p = jnp.min(jnp.where(keptv == 1, flat, 1024))
        p = jnp.where(p == 1024, _TOPK - 1, p)
        pm = flat == p
        gx1 = jnp.max(jnp.where(pm, ax1, _NEG))
        gy1 = jnp.max(jnp.where(pm, ay1, _NEG))
        gx2 = jnp.max(jnp.where(pm, ax2, _NEG))
        gy2 = jnp.max(jnp.where(pm, ay2, _NEG))
        gsc = jnp.max(jnp.where(pm, asc, _NEG))
        gcl = jnp.max(jnp.where(pm, acl, 0))
        tm = lane128 == t
        ox1 = jnp.where(tm, gx1, ox1)
        oy1 = jnp.where(tm, gy1, oy1)
        ox2 = jnp.where(tm, gx2, ox2)
        oy2 = jnp.where(tm, gy2, oy2)
        osc = jnp.where(tm, gsc, osc)
        ocl = jnp.where(tm, gcl, ocl)
        keptv = jnp.where(pm, 0, keptv)
        return keptv, ox1, oy1, ox2, oy2, osc, ocl

    def sel_body4(i, carry):
        carry = sel_body(4 * i, carry)
        carry = sel_body(4 * i + 1, carry)
        carry = sel_body(4 * i + 2, carry)
        carry = sel_body(4 * i + 3, carry)
        return carry

    zl = jnp.zeros((1, 128), jnp.float32)
    (_, ox1, oy1, ox2, oy2, osc, ocl) = jax.lax.fori_loop(
        0, _POSTK // 4, sel_body4,
        (kept, zl, zl, zl, zl, zl, jnp.zeros((1, 128), i32)))

    bT_ref[0:1, :] = ox1
    bT_ref[1:2, :] = oy1
    bT_ref[2:3, :] = ox2
    bT_ref[3:4, :] = oy2
    sc_ref[:, :] = osc
    cl_ref[:, :] = ocl


def _call(cls2d, ctr2d, reg2d, locT, interpret=False):
    return pl.pallas_call(
        _fcos_kernel,
        out_shape=[
            jax.ShapeDtypeStruct((4, 128), jnp.float32),
            jax.ShapeDtypeStruct((1, 128), jnp.float32),
            jax.ShapeDtypeStruct((1, 128), jnp.int32),
        ],
        scratch_shapes=[pltpu.VMEM((_C, _HW), jnp.float32)],
        interpret=interpret,
    )(cls2d, ctr2d, reg2d, locT)


@jax.jit
def kernel(locations, box_cls, box_regression, centerness):
    cls2d = box_cls.reshape(_C, _HW)
    reg2d = box_regression.reshape(4, _HW)
    ctr2d = centerness.reshape(1, _HW)
    locT = locations.T
    bT, sc, cl = _call(cls2d, ctr2d, reg2d, locT)
    boxes = bT[:, :_POSTK].T
    return boxes, sc[0, :_POSTK], cl[0, :_POSTK]


# speedup vs baseline: 2.9050x; 1.0314x over previous
import jax
import jax.numpy as jnp
from jax.experimental import pallas as pl
from jax.experimental.pallas import tpu as pltpu

_C = 80
_HW = 25600
_NCHUNK = 200
_TOPK = 1000
_POSTK = 100
_NEG = -3.4e38
_BIGI = 2**30


def _fcos_kernel(cls_ref, ctr_ref, reg_ref, loc_ref, bT_ref, sc_ref, cl_ref,
                 scores_ref):
    i32 = jnp.int32
    row80 = jax.lax.broadcasted_iota(i32, (_C, 128), 0)
    col80 = jax.lax.broadcasted_iota(i32, (_C, 128), 1)
    lane256 = jax.lax.broadcasted_iota(i32, (1, 256), 1)
    col4 = jax.lax.broadcasted_iota(i32, (4, 128), 1)
    col2 = jax.lax.broadcasted_iota(i32, (2, 128), 1)
    r8 = jax.lax.broadcasted_iota(i32, (8, 128), 0)
    c8 = jax.lax.broadcasted_iota(i32, (8, 128), 1)
    flat = r8 * 128 + c8

    def p1_body(c, carry):
        cmax, carg = carry
        base = pl.multiple_of(c * 128, 128)
        sg = jax.nn.sigmoid(cls_ref[:, pl.ds(base, 128)])
        ct = jax.nn.sigmoid(ctr_ref[:, pl.ds(base, 128)])
        s = jnp.where(sg > 0.05, sg * ct, -1.0)
        scores_ref[:, pl.ds(base, 128)] = s
        m = jnp.max(s)
        ridx = (base + col80) * _C + row80
        rmin = jnp.min(jnp.where(s == m, ridx, _BIGI))
        hit = lane256 == c
        return jnp.where(hit, m, cmax), jnp.where(hit, rmin, carg)

    cmax0 = jnp.full((1, 256), -3.0, jnp.float32)
    carg0 = jnp.full((1, 256), _BIGI, i32)
    cmax, carg = jax.lax.fori_loop(0, _NCHUNK, p1_body, (cmax0, carg0))

    def p2_body(t, carry):
        cmax, carg, m, r, ax1, ay1, ax2, ay2, asc, acl = carry
        loc = r // _C
        klass = r - loc * _C
        cc = loc // 128
        base = pl.multiple_of(cc * 128, 128)
        colin = loc - base
        blk = scores_ref[:, pl.ds(base, 128)]
        clr = jnp.where((row80 == klass) & (col80 == colin), -2.0, blk)
        scores_ref[:, pl.ds(base, 128)] = clr
        nm = jnp.max(clr)
        ridx = (base + col80) * _C + row80
        nr = jnp.min(jnp.where(clr == nm, ridx, _BIGI))
        lane_ne = lane256 != cc
        exm = jnp.max(jnp.where(lane_ne, cmax, -3.0))
        exr = jnp.min(jnp.where(lane_ne & (cmax == exm), carg, _BIGI))
        mp = jnp.maximum(exm, nm)
        rp = jnp.minimum(jnp.where(exm == mp, exr, _BIGI),
                         jnp.where(nm == mp, nr, _BIGI))
        hit = lane256 == cc
        cmax = jnp.where(hit, nm, cmax)
        carg = jnp.where(hit, nr, carg)
        rsel = jnp.where(col4 == colin, reg_ref[:, pl.ds(base, 128)], 0.0)
        lsel = jnp.where(col2 == colin, loc_ref[:, pl.ds(base, 128)], 0.0)
        r0 = jnp.sum(rsel[0:1, :])
        r1 = jnp.sum(rsel[1:2, :])
        r2 = jnp.sum(rsel[2:3, :])
        r3 = jnp.sum(rsel[3:4, :])
        lx = jnp.sum(lsel[0:1, :])
        ly = jnp.sum(lsel[1:2, :])
        pm = flat == t
        ax1 = jnp.where(pm, lx - r0, ax1)
        ay1 = jnp.where(pm, ly - r1, ay1)
        ax2 = jnp.where(pm, lx + r2, ax2)
        ay2 = jnp.where(pm, ly + r3, ay2)
        asc = jnp.where(pm, jnp.sqrt(m), asc)
        acl = jnp.where(pm, klass, acl)
        return cmax, carg, mp, rp, ax1, ay1, ax2, ay2, asc, acl

    def p2_body4(i, carry):
        carry = p2_body(4 * i, carry)
        carry = p2_body(4 * i + 1, carry)
        carry = p2_body(4 * i + 2, carry)
        carry = p2_body(4 * i + 3, carry)
        return carry

    m0 = jnp.max(cmax)
    r0_ = jnp.min(jnp.where(cmax == m0, carg, _BIGI))
    zf = jnp.zeros((8, 128), jnp.float32)
    init = (cmax, carg, m0, r0_, zf, zf, zf, zf,
            jnp.full((8, 128), -1.0, jnp.float32), jnp.zeros((8, 128), i32))
    (_, _, _, _, ax1, ay1, ax2, ay2, asc, acl) = jax.lax.fori_loop(
        0, _TOPK // 4, p2_body4, init)

    valid = flat < _TOPK
    mc = jnp.max(jnp.where(
        valid,
        jnp.maximum(jnp.maximum(ax1, ax2), jnp.maximum(ay1, ay2)), _NEG))
    off = acl.astype(jnp.float32) * (mc + 1.0)
    bx1 = ax1 + off
    by1 = ay1 + off
    bx2 = ax2 + off
    by2 = ay2 + off
    area = jnp.maximum(bx2 - bx1, 0.0) * jnp.maximum(by2 - by1, 0.0)

    def nms_body(pos, supp):
        pm = flat == pos
        px1 = jnp.max(jnp.where(pm, bx1, _NEG))
        py1 = jnp.max(jnp.where(pm, by1, _NEG))
        px2 = jnp.max(jnp.where(pm, bx2, _NEG))
        py2 = jnp.max(jnp.where(pm, by2, _NEG))
        pa = jnp.max(jnp.where(pm, area, _NEG))
        alive = jnp.max(jnp.where(pm, supp, 0)) == 0
        ix1 = jnp.maximum(px1, bx1)
        iy1 = jnp.maximum(py1, by1)
        ix2 = jnp.minimum(px2, bx2)
        iy2 = jnp.minimum(py2, by2)
        inter = jnp.maximum(ix2 - ix1, 0.0) * jnp.maximum(iy2 - iy1, 0.0)
        iou = inter / (pa + area - inter + 1e-9)
        supnew = alive & (flat > pos) & (iou > 0.6)
        return jnp.where(supnew, 1, supp)

    def nms_body4(i, supp):
        supp = nms_body(4 * i, supp)
        supp = nms_body(4 * i + 1, supp)
        supp = nms_body(4 * i + 2, supp)
        supp = nms_body(4 * i + 3, supp)
        return supp

    supp = jax.lax.fori_loop(0, _TOPK // 4, nms_body4,
                             jnp.where(valid, 0, 1).astype(i32))
    kept = jnp.where((supp == 0) & valid, 1, 0).astype(i32)

    lane128 = jax.lax.broadcasted_iota(i32, (1, 128), 1)

    def sel_body(t, carry):
        keptv, ox1, oy1, ox2, oy2, osc, ocl = carry
        p = jnp.min(jnp.where(keptv == 1, flat, 1024))
        p = jnp.where(p == 1024, _TOPK - 1, p)
        pm = flat == p
        gx1 = jnp.max(jnp.where(pm, ax1, _NEG))
        gy1 = jnp.max(jnp.where(pm, ay1, _NEG))
        gx2 = jnp.max(jnp.where(pm, ax2, _NEG))
        gy2 = jnp.max(jnp.where(pm, ay2, _NEG))
        gsc = jnp.max(jnp.where(pm, asc, _NEG))
        gcl = jnp.max(jnp.where(pm, acl, 0))
        tm = lane128 == t
        ox1 = jnp.where(tm, gx1, ox1)
        oy1 = jnp.where(tm, gy1, oy1)
        ox2 = jnp.where(tm, gx2, ox2)
        oy2 = jnp.where(tm, gy2, oy2)
        osc = jnp.where(tm, gsc, osc)
        ocl = jnp.where(tm, gcl, ocl)
        keptv = jnp.where(pm, 0, keptv)
        return keptv, ox1, oy1, ox2, oy2, osc, ocl

    def sel_body4(i, carry):
        carry = sel_body(4 * i, carry)
        carry = sel_body(4 * i + 1, carry)
        carry = sel_body(4 * i + 2, carry)
        carry = sel_body(4 * i + 3, carry)
        return carry

    zl = jnp.zeros((1, 128), jnp.float32)
    (_, ox1, oy1, ox2, oy2, osc, ocl) = jax.lax.fori_loop(
        0, _POSTK // 4, sel_body4,
        (kept, zl, zl, zl, zl, zl, jnp.zeros((1, 128), i32)))

    bT_ref[0:1, :] = ox1
    bT_ref[1:2, :] = oy1
    bT_ref[2:3, :] = ox2
    bT_ref[3:4, :] = oy2
    sc_ref[:, :] = osc
    cl_ref[:, :] = ocl


def _call(cls2d, ctr2d, reg2d, locT, interpret=False):
    return pl.pallas_call(
        _fcos_kernel,
        out_shape=[
            jax.ShapeDtypeStruct((4, 128), jnp.float32),
            jax.ShapeDtypeStruct((1, 128), jnp.float32),
            jax.ShapeDtypeStruct((1, 128), jnp.int32),
        ],
        scratch_shapes=[pltpu.VMEM((_C, _HW), jnp.float32)],
        interpret=interpret,
    )(cls2d, ctr2d, reg2d, locT)


@jax.jit
def kernel(locations, box_cls, box_regression, centerness):
    cls2d = box_cls.reshape(_C, _HW)
    reg2d = box_regression.reshape(4, _HW)
    ctr2d = centerness.reshape(1, _HW)
    locT = locations.T
    bT, sc, cl = _call(cls2d, ctr2d, reg2d, locT)
    boxes = bT[:, :_POSTK].T
    return boxes, sc[0, :_POSTK], cl[0, :_POSTK]
